# baseline (speedup 1.0000x reference)
import functools

import jax
import jax.numpy as jnp
from jax import lax
from jax.experimental import pallas as pl
from jax.experimental.pallas import tpu as pltpu
from jax.experimental.pallas import tpu_sc as plsc

NC = 2
NS = 16
NW = NC * NS
CH = 128
LANES = 16
NBUF = 5


def _make_body(total, S, D):
    per_w = total // NW
    n_chunks = per_w // CH
    n_col = D // LANES
    assert n_chunks >= 2 * NBUF and (n_chunks - 2 * NBUF) % NBUF == 0

    def body(x_hbm, posdup_hbm, tok_hbm, out_hbm,
             pos_v, idx_v, rows_v, out_v, gsems, osems):
        wid = lax.axis_index("s") * NC + lax.axis_index("c")
        base0 = wid * per_w
        pltpu.sync_copy(posdup_hbm, pos_v)

        def gather(k, slot):
            base = base0 + k * CH
            return pltpu.make_async_copy(
                tok_hbm.at[idx_v.at[slot]], rows_v.at[slot], gsems.at[slot])

        def store(k, slot):
            base = base0 + k * CH
            return pltpu.make_async_copy(
                out_v.at[slot], out_hbm.at[pl.ds(base, CH)], osems.at[slot])

        def start_chunk(k, slot):
            base = base0 + k * CH
            pltpu.sync_copy(x_hbm.at[pl.ds(base, CH)], idx_v.at[slot])
            gather(k, slot).start()

        def compute_chunk(k, slot):
            p0 = lax.rem(base0 + k * CH, S)

            def row_body(j, c2):
                r0 = j * LANES
                for i in range(LANES):
                    r = r0 + i
                    pr = p0 + r
                    for c in range(n_col):
                        sl = pl.ds(c * LANES, LANES)
                        out_v[slot, r, sl] = rows_v[slot, r, sl] + pos_v[pr, sl]
                return c2

            lax.fori_loop(0, CH // LANES, row_body, 0)

        for s in range(NBUF):
            start_chunk(s, s)
        for k in range(NBUF):
            gather(k, k).wait()
            compute_chunk(k, k)
            store(k, k).start()
            start_chunk(k + NBUF, k)

        def main_body(k2, c2):
            for b in range(NBUF):
                k = NBUF + k2 * NBUF + b
                gather(k, b).wait()
                store(k - NBUF, b).wait()
                compute_chunk(k, b)
                store(k, b).start()
                start_chunk(k + NBUF, b)
            return c2

        lax.fori_loop(0, (n_chunks - 2 * NBUF) // NBUF, main_body, 0)

        for k in range(n_chunks - NBUF, n_chunks):
            slot = k % NBUF
            gather(k, slot).wait()
            store(k - NBUF, slot).wait()
            compute_chunk(k, slot)
            store(k, slot).start()
        for k in range(n_chunks - NBUF, n_chunks):
            store(k, k % NBUF).wait()

    return body


@functools.partial(jax.jit, static_argnames=())
def kernel(x, tok_table, pos_table):
    B, S = x.shape
    V, D = tok_table.shape
    total = B * S
    xf = x.reshape(total).astype(jnp.int32)
    posdup = jnp.concatenate([pos_table, pos_table], axis=0)

    mesh = plsc.VectorSubcoreMesh(core_axis_name="c", subcore_axis_name="s")
    run = pl.kernel(
        _make_body(total, S, D),
        mesh=mesh,
        compiler_params=pltpu.CompilerParams(use_tc_tiling_on_sc=False),
        out_type=jax.ShapeDtypeStruct((total, D), jnp.float32),
        scratch_types=[
            pltpu.VMEM((2 * S, D), jnp.float32),
            pltpu.VMEM((NBUF, CH), jnp.int32),
            pltpu.VMEM((NBUF, CH, D), jnp.float32),
            pltpu.VMEM((NBUF, CH, D), jnp.float32),
            pltpu.SemaphoreType.DMA((NBUF,)),
            pltpu.SemaphoreType.DMA((NBUF,)),
        ],
    )
    out = run(xf, posdup, tok_table)
    return out.reshape(B, S, D)


# speedup vs baseline: 1.0241x; 1.0241x over previous
import functools

import jax
import jax.numpy as jnp
from jax import lax
from jax.experimental import pallas as pl
from jax.experimental.pallas import tpu as pltpu
from jax.experimental.pallas import tpu_sc as plsc

NC = 2
NS = 16
NW = NC * NS
CH = 128
LANES = 16
NBUF = 5


def _make_body(total, S, D):
    per_w = total // NW
    n_chunks = per_w // CH
    n_col = D // LANES
    assert n_chunks >= 2 * NBUF and (n_chunks - 2 * NBUF) % NBUF == 0

    def body(x_hbm, posdup_hbm, tok_hbm, out_hbm,
             pos_v, idx_v, rows_v, out_v, gsems, osems):
        wid = lax.axis_index("s") * NC + lax.axis_index("c")
        base0 = wid * per_w
        pltpu.sync_copy(posdup_hbm, pos_v)
        pltpu.sync_copy(x_hbm.at[pl.ds(base0, per_w)], idx_v)

        def gather(k, slot):
            return pltpu.make_async_copy(
                tok_hbm.at[idx_v.at[pl.ds(k * CH, CH)]], rows_v.at[slot],
                gsems.at[slot])

        def store(k, slot):
            base = base0 + k * CH
            return pltpu.make_async_copy(
                out_v.at[slot], out_hbm.at[pl.ds(base, CH)], osems.at[slot])

        def start_chunk(k, slot):
            gather(k, slot).start()

        def compute_chunk(k, slot):
            p0 = lax.rem(base0 + k * CH, S)

            def row_body(j, c2):
                r0 = j * LANES
                for i in range(LANES):
                    r = r0 + i
                    pr = p0 + r
                    for c in range(n_col):
                        sl = pl.ds(c * LANES, LANES)
                        out_v[slot, r, sl] = rows_v[slot, r, sl] + pos_v[pr, sl]
                return c2

            lax.fori_loop(0, CH // LANES, row_body, 0)

        for s in range(NBUF):
            start_chunk(s, s)
        for k in range(NBUF):
            gather(k, k).wait()
            compute_chunk(k, k)
            store(k, k).start()
            start_chunk(k + NBUF, k)

        def main_body(k2, c2):
            for b in range(NBUF):
                k = NBUF + k2 * NBUF + b
                gather(k, b).wait()
                store(k - NBUF, b).wait()
                compute_chunk(k, b)
                store(k, b).start()
                start_chunk(k + NBUF, b)
            return c2

        lax.fori_loop(0, (n_chunks - 2 * NBUF) // NBUF, main_body, 0)

        for k in range(n_chunks - NBUF, n_chunks):
            slot = k % NBUF
            gather(k, slot).wait()
            store(k - NBUF, slot).wait()
            compute_chunk(k, slot)
            store(k, slot).start()
        for k in range(n_chunks - NBUF, n_chunks):
            store(k, k % NBUF).wait()

    return body


@functools.partial(jax.jit, static_argnames=())
def kernel(x, tok_table, pos_table):
    B, S = x.shape
    V, D = tok_table.shape
    total = B * S
    xf = x.reshape(total).astype(jnp.int32)
    posdup = jnp.concatenate([pos_table, pos_table], axis=0)

    mesh = plsc.VectorSubcoreMesh(core_axis_name="c", subcore_axis_name="s")
    run = pl.kernel(
        _make_body(total, S, D),
        mesh=mesh,
        compiler_params=pltpu.CompilerParams(use_tc_tiling_on_sc=False),
        out_type=jax.ShapeDtypeStruct((total, D), jnp.float32),
        scratch_types=[
            pltpu.VMEM((2 * S, D), jnp.float32),
            pltpu.VMEM((total // NW,), jnp.int32),
            pltpu.VMEM((NBUF, CH, D), jnp.float32),
            pltpu.VMEM((NBUF, CH, D), jnp.float32),
            pltpu.SemaphoreType.DMA((NBUF,)),
            pltpu.SemaphoreType.DMA((NBUF,)),
        ],
    )
    out = run(xf, posdup, tok_table)
    return out.reshape(B, S, D)
